# baseline (speedup 1.0000x reference)
import functools

import jax
import jax.numpy as jnp
from jax.experimental import pallas as pl
from jax.experimental.pallas import tpu as pltpu

B, HEADS, L, DIM, CODES = 4, 16, 4096, 64, 128
TL = 1024


def _fused_body(x_ref, c_ref, out_ref):
    sim = jax.lax.dot_general(
        x_ref[...], c_ref[...],
        dimension_numbers=(((1,), (1,)), ((), ())),
        preferred_element_type=jnp.float32,
    )
    idx = jnp.argmax(sim, axis=-1).astype(jnp.int32)
    iota = jax.lax.broadcasted_iota(jnp.int32, (TL, CODES), 1)
    out_ref[...] = (iota == idx[:, None]).astype(jnp.float32)


@functools.partial(jax.jit, static_argnames=("interpret",))
def _fused_call(x, c, interpret=False):
    grid = (HEADS, B, L // TL)
    out = pl.pallas_call(
        _fused_body,
        grid=grid,
        in_specs=[
            pl.BlockSpec((None, None, TL, DIM), lambda h, b, j: (b, h, j, 0)),
            pl.BlockSpec((None, CODES, DIM), lambda h, b, j: (h, 0, 0)),
        ],
        out_specs=pl.BlockSpec((None, None, TL, CODES),
                               lambda h, b, j: (b, h, j, 0)),
        out_shape=jax.ShapeDtypeStruct((B, HEADS, L, CODES), jnp.float32),
        interpret=interpret,
    )(x, c)
    return out


def kernel(x, c):
    onehot = _fused_call(x, c)
    return (onehot, c)


# speedup vs baseline: 1.2863x; 1.2863x over previous
import functools

import jax
import jax.numpy as jnp
from jax.experimental import pallas as pl
from jax.experimental.pallas import tpu as pltpu

B, HEADS, L, DIM, CODES = 4, 16, 4096, 64, 128
TL = 2048


def _fused_body(x_ref, c_ref, out_ref):
    sim = jax.lax.dot_general(
        x_ref[...], c_ref[...],
        dimension_numbers=(((1,), (1,)), ((), ())),
        preferred_element_type=jnp.float32,
    )
    idx = jnp.argmax(sim, axis=-1).astype(jnp.int32)
    iota = jax.lax.broadcasted_iota(jnp.int32, (TL, CODES), 1)
    out_ref[...] = (iota == idx[:, None]).astype(jnp.float32)


@functools.partial(jax.jit, static_argnames=("interpret",))
def _fused_call(x, c, interpret=False):
    grid = (HEADS, B, L // TL)
    out = pl.pallas_call(
        _fused_body,
        grid=grid,
        in_specs=[
            pl.BlockSpec((None, None, TL, DIM), lambda h, b, j: (b, h, j, 0)),
            pl.BlockSpec((None, CODES, DIM), lambda h, b, j: (h, 0, 0)),
        ],
        out_specs=pl.BlockSpec((None, None, TL, CODES),
                               lambda h, b, j: (b, h, j, 0)),
        out_shape=jax.ShapeDtypeStruct((B, HEADS, L, CODES), jnp.float32),
        compiler_params=pltpu.CompilerParams(
            dimension_semantics=("parallel", "parallel", "arbitrary")),
        interpret=interpret,
    )(x, c)
    return out


def kernel(x, c):
    onehot = _fused_call(x, c)
    return (onehot, c)
